# baseline (speedup 1.0000x reference)
import functools

import jax
import jax.numpy as jnp
from jax import lax
from jax.experimental import pallas as pl
from jax.experimental.pallas import tpu as pltpu
from jax.experimental.pallas import tpu_sc as plsc

N = 10000
E = 320000
D = 128

LANES = 128
NCORE = 2
NSUB = 16
NTILE = NCORE * NSUB
NBLK = E // LANES
SBLK = -(-NBLK // NTILE)

ROWS_PER_SUB = 624
TAIL = N - NSUB * ROWS_PER_SUB

_DOT_KW = dict(preferred_element_type=jnp.float32,
               precision=lax.Precision.HIGHEST)


def _matT(x, w):
    return lax.dot_general(x, w, (((1,), (1,)), ((), ())), **_DOT_KW)


def _tc_a_body(x_ref, wl_ref, wr_ref, y_ref, r_ref):
    x = x_ref[...]
    y_ref[...] = _matT(x, wl_ref[...])
    r_ref[...] = _matT(x, wr_ref[...])


def _tc_a(x, wl, wr):
    return pl.pallas_call(
        _tc_a_body,
        out_shape=[jax.ShapeDtypeStruct((N, D), jnp.float32),
                   jax.ShapeDtypeStruct((N, D), jnp.float32)],
    )(x, wl, wr)


def _tc_b_body(p_ref, d_ref, r_ref, b_ref, w2l_ref, w2r_ref,
               y2_ref, r2_ref, inv_ref):
    agg = p_ref[0] + p_ref[1]
    deg = d_ref[0] + d_ref[1]
    inv = 1.0 / jnp.maximum(deg, 1.0)
    h = jnp.maximum(agg * inv[:, None] + b_ref[...][None, :] + r_ref[...], 0.0)
    y2_ref[...] = _matT(h, w2l_ref[...])
    r2_ref[...] = _matT(h, w2r_ref[...])
    inv_ref[...] = inv


def _tc_b(p, d, r1, b1, w2l, w2r):
    return pl.pallas_call(
        _tc_b_body,
        out_shape=[jax.ShapeDtypeStruct((N, D), jnp.float32),
                   jax.ShapeDtypeStruct((N, D), jnp.float32),
                   jax.ShapeDtypeStruct((N,), jnp.float32)],
    )(p, d, r1, b1, w2l, w2r)


def _tc_c_body(p_ref, inv_ref, r_ref, b_ref, o_ref):
    agg = p_ref[0] + p_ref[1]
    o_ref[...] = agg * inv_ref[...][:, None] + b_ref[...][None, :] + r_ref[...]


def _tc_c(p, inv, r2, b2):
    return pl.pallas_call(
        _tc_c_body,
        out_shape=jax.ShapeDtypeStruct((N, D), jnp.float32),
    )(p, inv, r2, b2)


def _sc_agg_kernel(with_deg):
    mesh = plsc.VectorSubcoreMesh(core_axis_name="c", subcore_axis_name="s")
    out_type = [jax.ShapeDtypeStruct((NCORE, N, D), jnp.float32)]
    scratch = [
        pltpu.VMEM((SBLK * LANES,), jnp.int32),
        pltpu.VMEM((SBLK * LANES,), jnp.int32),
        pltpu.VMEM((LANES,), jnp.int32),
        pltpu.VMEM((LANES, D), jnp.float32),
        pltpu.VMEM_SHARED((N, D), jnp.float32),
        pltpu.SemaphoreType.DMA,
    ]
    if with_deg:
        out_type.append(jax.ShapeDtypeStruct((NCORE * N,), jnp.float32))
        scratch += [
            pltpu.VMEM((LANES,), jnp.float32),
            pltpu.VMEM_SHARED((N,), jnp.float32),
            pltpu.VMEM((ROWS_PER_SUB,), jnp.float32),
        ]

    def body(*refs):
        if with_deg:
            (y_hbm, src_hbm, dst_hbm, z2_hbm, out_p, out_d,
             src_v, dst_v, dblk_v, rows_v, agg_sh, gsem,
             ones_v, deg_sh, dbounce) = refs
        else:
            (y_hbm, src_hbm, dst_hbm, z2_hbm, out_p,
             src_v, dst_v, dblk_v, rows_v, agg_sh, gsem) = refs
        c = lax.axis_index("c")
        s = lax.axis_index("s")
        w = c * NSUB + s

        pltpu.sync_copy(z2_hbm.at[pl.ds(s * ROWS_PER_SUB, ROWS_PER_SUB)],
                        agg_sh.at[pl.ds(s * ROWS_PER_SUB, ROWS_PER_SUB)])
        if with_deg:
            for j in range(LANES // 16):
                ones_v[pl.ds(j * 16, 16)] = jnp.full((16,), 1.0, jnp.float32)
            for j in range(ROWS_PER_SUB // 16):
                dbounce[pl.ds(j * 16, 16)] = jnp.zeros((16,), jnp.float32)
            pltpu.sync_copy(dbounce,
                            deg_sh.at[pl.ds(s * ROWS_PER_SUB, ROWS_PER_SUB)])

        @pl.when(s == 0)
        def _zero_tail():
            pltpu.sync_copy(z2_hbm.at[pl.ds(NSUB * ROWS_PER_SUB, TAIL)],
                            agg_sh.at[pl.ds(NSUB * ROWS_PER_SUB, TAIL)])
            if with_deg:
                pltpu.sync_copy(dbounce.at[pl.ds(0, TAIL)],
                                deg_sh.at[pl.ds(NSUB * ROWS_PER_SUB, TAIL)])

        start_b = (w * NBLK) // NTILE
        end_b = ((w + 1) * NBLK) // NTILE
        base_b = jnp.minimum(start_b, NBLK - SBLK)
        pltpu.sync_copy(src_hbm.at[pl.ds(base_b * LANES, SBLK * LANES)],
                        src_v)
        pltpu.sync_copy(dst_hbm.at[pl.ds(base_b * LANES, SBLK * LANES)],
                        dst_v)

        plsc.subcore_barrier()

        nsplit = 2
        part = LANES // nsplit
        def blk(i, carry):
            gs = [pltpu.async_copy(
                y_hbm.at[src_v.at[pl.ds(i * LANES + k * part, part)]],
                rows_v.at[pl.ds(k * part, part)], gsem)
                for k in range(nsplit)]
            for j in range(LANES // 16):
                dblk_v[pl.ds(j * 16, 16)] = dst_v[pl.ds(i * LANES + j * 16,
                                                        16)]
            for g in gs:
                g.wait()
            pltpu.sync_copy(rows_v, agg_sh.at[dblk_v], add=True)
            if with_deg:
                pltpu.sync_copy(ones_v, deg_sh.at[dblk_v], add=True)
            return carry

        lax.fori_loop(start_b - base_b, end_b - base_b, blk, 0)

        plsc.subcore_barrier()

        pltpu.sync_copy(agg_sh.at[pl.ds(s * ROWS_PER_SUB, ROWS_PER_SUB)],
                        out_p.at[c, pl.ds(s * ROWS_PER_SUB, ROWS_PER_SUB)])
        if with_deg:
            pltpu.sync_copy(deg_sh.at[pl.ds(s * ROWS_PER_SUB, ROWS_PER_SUB)],
                            dbounce)
            pltpu.sync_copy(dbounce,
                            out_d.at[pl.ds(c * N + s * ROWS_PER_SUB,
                                           ROWS_PER_SUB)])

        @pl.when(s == 0)
        def _copy_tail():
            pltpu.sync_copy(agg_sh.at[pl.ds(NSUB * ROWS_PER_SUB, TAIL)],
                            out_p.at[c, pl.ds(NSUB * ROWS_PER_SUB, TAIL)])
            if with_deg:
                pltpu.sync_copy(deg_sh.at[pl.ds(NSUB * ROWS_PER_SUB, TAIL)],
                                dbounce.at[pl.ds(0, TAIL)])
                pltpu.sync_copy(dbounce.at[pl.ds(0, TAIL)],
                                out_d.at[pl.ds(c * N + NSUB * ROWS_PER_SUB,
                                               TAIL)])

    return pl.kernel(body, out_type=out_type, mesh=mesh,
                     scratch_types=scratch)


_sc_agg_deg = _sc_agg_kernel(with_deg=True)
_sc_agg = _sc_agg_kernel(with_deg=False)


def kernel(x, edge_index, W1l, b1l, W1r, W2l, b2l, W2r):
    ei = edge_index.astype(jnp.int32)
    src = ei[0]
    dst = ei[1]
    z2 = jnp.zeros((N, D), jnp.float32)

    y1, r1 = _tc_a(x, W1l, W1r)
    p1, d1 = _sc_agg_deg(y1, src, dst, z2)
    y2, r2, inv = _tc_b(p1, d1.reshape(NCORE, N), r1, b1l, W2l, W2r)
    (p2,) = _sc_agg(y2, src, dst, z2)
    return _tc_c(p2, inv, r2, b2l)


# speedup vs baseline: 1.0012x; 1.0012x over previous
import jax
import jax.numpy as jnp
from jax import lax
from jax.experimental import pallas as pl
from jax.experimental.pallas import tpu as pltpu
from jax.experimental.pallas import tpu_sc as plsc

N = 10000
E = 320000
D = 128

LANES = 128
NCORE = 2
NSUB = 16
NTILE = NCORE * NSUB
NBLK = E // LANES
SBLK = -(-NBLK // NTILE)

ROWS_PER_SUB = 624
TAIL = N - NSUB * ROWS_PER_SUB

_DOT_KW = dict(preferred_element_type=jnp.float32,
               precision=lax.Precision.HIGHEST)


def _matT(x, w):
    return lax.dot_general(x, w, (((1,), (1,)), ((), ())), **_DOT_KW)


def _tc_a_body(x_ref, wl_ref, wr_ref, y_ref, r_ref):
    x = x_ref[...]
    y_ref[...] = _matT(x, wl_ref[...])
    r_ref[...] = _matT(x, wr_ref[...])


def _tc_a(x, wl, wr):
    return pl.pallas_call(
        _tc_a_body,
        out_shape=[jax.ShapeDtypeStruct((N, D), jnp.float32),
                   jax.ShapeDtypeStruct((N, D), jnp.float32)],
    )(x, wl, wr)


def _tc_b_body(p_ref, d_ref, r_ref, b_ref, w2l_ref, w2r_ref,
               y2_ref, r2_ref, inv_ref):
    agg = p_ref[0] + p_ref[1]
    deg = d_ref[0] + d_ref[1]
    inv = 1.0 / jnp.maximum(deg, 1.0)
    h = jnp.maximum(agg * inv[:, None] + b_ref[...][None, :] + r_ref[...], 0.0)
    y2_ref[...] = _matT(h, w2l_ref[...])
    r2_ref[...] = _matT(h, w2r_ref[...])
    inv_ref[...] = inv


def _tc_b(p, d, r1, b1, w2l, w2r):
    return pl.pallas_call(
        _tc_b_body,
        out_shape=[jax.ShapeDtypeStruct((N, D), jnp.float32),
                   jax.ShapeDtypeStruct((N, D), jnp.float32),
                   jax.ShapeDtypeStruct((N,), jnp.float32)],
    )(p, d, r1, b1, w2l, w2r)


def _tc_c_body(p_ref, inv_ref, r_ref, b_ref, o_ref):
    agg = p_ref[0] + p_ref[1]
    o_ref[...] = agg * inv_ref[...][:, None] + b_ref[...][None, :] + r_ref[...]


def _tc_c(p, inv, r2, b2):
    return pl.pallas_call(
        _tc_c_body,
        out_shape=jax.ShapeDtypeStruct((N, D), jnp.float32),
    )(p, inv, r2, b2)


def _sc_agg_kernel(with_deg):
    mesh = plsc.VectorSubcoreMesh(core_axis_name="c", subcore_axis_name="s")
    out_type = [jax.ShapeDtypeStruct((NCORE, N, D), jnp.float32)]
    scratch = [
        pltpu.VMEM((SBLK * LANES,), jnp.int32),
        pltpu.VMEM((SBLK * LANES,), jnp.int32),
        pltpu.VMEM((LANES,), jnp.int32),
        pltpu.VMEM((LANES, D), jnp.float32),
        pltpu.VMEM_SHARED((N, D), jnp.float32),
        pltpu.SemaphoreType.DMA,
    ]
    if with_deg:
        out_type.append(jax.ShapeDtypeStruct((NCORE * N,), jnp.float32))
        scratch += [
            pltpu.VMEM((LANES,), jnp.float32),
            pltpu.VMEM_SHARED((N,), jnp.float32),
            pltpu.VMEM((ROWS_PER_SUB,), jnp.float32),
        ]

    def body(*refs):
        if with_deg:
            (y_hbm, src_hbm, dst_hbm, z2_hbm, out_p, out_d,
             src_v, dst_v, dblk_v, rows_v, agg_sh, gsem,
             ones_v, deg_sh, dbounce) = refs
        else:
            (y_hbm, src_hbm, dst_hbm, z2_hbm, out_p,
             src_v, dst_v, dblk_v, rows_v, agg_sh, gsem) = refs
        c = lax.axis_index("c")
        s = lax.axis_index("s")
        w = c * NSUB + s

        pltpu.sync_copy(z2_hbm.at[pl.ds(s * ROWS_PER_SUB, ROWS_PER_SUB)],
                        agg_sh.at[pl.ds(s * ROWS_PER_SUB, ROWS_PER_SUB)])
        if with_deg:
            for j in range(LANES // 16):
                ones_v[pl.ds(j * 16, 16)] = jnp.full((16,), 1.0, jnp.float32)
            for j in range(ROWS_PER_SUB // 16):
                dbounce[pl.ds(j * 16, 16)] = jnp.zeros((16,), jnp.float32)
            pltpu.sync_copy(dbounce,
                            deg_sh.at[pl.ds(s * ROWS_PER_SUB, ROWS_PER_SUB)])

        @pl.when(s == 0)
        def _zero_tail():
            pltpu.sync_copy(z2_hbm.at[pl.ds(NSUB * ROWS_PER_SUB, TAIL)],
                            agg_sh.at[pl.ds(NSUB * ROWS_PER_SUB, TAIL)])
            if with_deg:
                pltpu.sync_copy(dbounce.at[pl.ds(0, TAIL)],
                                deg_sh.at[pl.ds(NSUB * ROWS_PER_SUB, TAIL)])

        start_b = (w * NBLK) // NTILE
        end_b = ((w + 1) * NBLK) // NTILE
        base_b = jnp.minimum(start_b, NBLK - SBLK)
        pltpu.sync_copy(src_hbm.at[pl.ds(base_b * LANES, SBLK * LANES)],
                        src_v)
        pltpu.sync_copy(dst_hbm.at[pl.ds(base_b * LANES, SBLK * LANES)],
                        dst_v)

        plsc.subcore_barrier()

        nsplit = 2
        part = LANES // nsplit
        def blk(i, carry):
            gs = [pltpu.async_copy(
                y_hbm.at[src_v.at[pl.ds(i * LANES + k * part, part)]],
                rows_v.at[pl.ds(k * part, part)], gsem)
                for k in range(nsplit)]
            for j in range(LANES // 16):
                dblk_v[pl.ds(j * 16, 16)] = dst_v[pl.ds(i * LANES + j * 16,
                                                        16)]
            for g in gs:
                g.wait()
            pltpu.sync_copy(rows_v, agg_sh.at[dblk_v], add=True)
            if with_deg:
                pltpu.sync_copy(ones_v, deg_sh.at[dblk_v], add=True)
            return carry

        lax.fori_loop(start_b - base_b, end_b - base_b, blk, 0)

        plsc.subcore_barrier()

        pltpu.sync_copy(agg_sh.at[pl.ds(s * ROWS_PER_SUB, ROWS_PER_SUB)],
                        out_p.at[c, pl.ds(s * ROWS_PER_SUB, ROWS_PER_SUB)])
        if with_deg:
            pltpu.sync_copy(deg_sh.at[pl.ds(s * ROWS_PER_SUB, ROWS_PER_SUB)],
                            dbounce)
            pltpu.sync_copy(dbounce,
                            out_d.at[pl.ds(c * N + s * ROWS_PER_SUB,
                                           ROWS_PER_SUB)])

        @pl.when(s == 0)
        def _copy_tail():
            pltpu.sync_copy(agg_sh.at[pl.ds(NSUB * ROWS_PER_SUB, TAIL)],
                            out_p.at[c, pl.ds(NSUB * ROWS_PER_SUB, TAIL)])
            if with_deg:
                pltpu.sync_copy(deg_sh.at[pl.ds(NSUB * ROWS_PER_SUB, TAIL)],
                                dbounce.at[pl.ds(0, TAIL)])
                pltpu.sync_copy(dbounce.at[pl.ds(0, TAIL)],
                                out_d.at[pl.ds(c * N + NSUB * ROWS_PER_SUB,
                                               TAIL)])

    return pl.kernel(body, out_type=out_type, mesh=mesh,
                     scratch_types=scratch)


_sc_agg_deg = _sc_agg_kernel(with_deg=True)
_sc_agg = _sc_agg_kernel(with_deg=False)


def kernel(x, edge_index, W1l, b1l, W1r, W2l, b2l, W2r):
    ei = edge_index.astype(jnp.int32)
    src = ei[0]
    dst = ei[1]
    z2 = jnp.zeros((N, D), jnp.float32)

    y1, r1 = _tc_a(x, W1l, W1r)
    p1, d1 = _sc_agg_deg(y1, src, dst, z2)
    y2, r2, inv = _tc_b(p1, d1.reshape(NCORE, N), r1, b1l, W2l, W2r)
    (p2,) = _sc_agg(y2, src, dst, z2)
    return _tc_c(p2, inv, r2, b2l)
